# baseline (speedup 1.0000x reference)
import functools

import jax
import jax.numpy as jnp
import numpy as np
from jax import lax
from jax.experimental import pallas as pl
from jax.experimental.pallas import tpu as pltpu
from jax.experimental.pallas import tpu_sc as plsc

N = 8192
D = 768
E = 8
K = 2
DCSI = 10
DFF = 768
BALANCE_WEIGHT = 0.5

P = N * K
BA = 512
NTA = N // BA
BM = 256
M = P + E * BM
NB = M // BM
BTB = 2048
NTB = N // BTB

NWORK = 32
L = 16
HD = D // 2


def _bf_to_i32(a):
    n = a.shape[0]
    return jax.lax.bitcast_convert_type(a.reshape(n, HD, 2), jnp.int32)


def _i32_to_bf(a):
    n = a.shape[0]
    return jax.lax.bitcast_convert_type(a, jnp.bfloat16).reshape(n, D)


def _router_kernel(h_ref, csi_ref, rwh_ref, rwc_ref, rb_ref, tri_ref,
                   e0_ref, e1_ref, g0_ref, g1_ref, r0_ref, r1_ref,
                   cnt_ref, loss_ref, hbf_ref,
                   oh2_ref, base_ref, sump_ref):
    i = pl.program_id(0)

    @pl.when(i == 0)
    def _init():
        base_ref[...] = jnp.zeros_like(base_ref)
        sump_ref[...] = jnp.zeros_like(sump_ref)

    @pl.when(i < NTA)
    def _phase1():
        t = i
        logits = (jnp.dot(h_ref[...], rwh_ref[...],
                          preferred_element_type=jnp.float32)
                  + jnp.dot(csi_ref[...], rwc_ref[...],
                            preferred_element_type=jnp.float32)
                  + rb_ref[...])
        mx = jnp.max(logits, axis=-1, keepdims=True)
        ex = jnp.exp(logits - mx)
        probs = ex / jnp.sum(ex, axis=-1, keepdims=True)
        lane = jax.lax.broadcasted_iota(jnp.int32, (BA, E), 1)
        v1 = jnp.max(probs, axis=-1, keepdims=True)
        e1 = jnp.min(jnp.where(probs == v1, lane, E), axis=-1, keepdims=True)
        masked = jnp.where(lane == e1, -jnp.inf, probs)
        v2 = jnp.max(masked, axis=-1, keepdims=True)
        e2 = jnp.min(jnp.where(masked == v2, lane, E), axis=-1, keepdims=True)
        denom = v1 + v2
        oh1 = (lane == e1).astype(jnp.float32)
        oh2 = (lane == e2).astype(jnp.float32)

        e0_ref[...] = e1
        e1_ref[...] = e2
        g0_ref[...] = v1 / denom
        g1_ref[...] = v2 / denom
        hbf_ref[...] = h_ref[...].astype(jnp.bfloat16)
        oh2_ref[pl.ds(t * BA, BA), :] = oh2

        cum = base_ref[...] + jnp.dot(tri_ref[...], oh1,
                                      preferred_element_type=jnp.float32)
        r0_ref[...] = jnp.sum(oh1 * cum, axis=-1, keepdims=True)
        base_ref[...] += jnp.sum(oh1, axis=0, keepdims=True)
        sump_ref[...] += jnp.sum(probs, axis=0, keepdims=True)

    @pl.when(i >= NTA)
    def _phase2():
        t = i - NTA
        oh2 = oh2_ref[pl.ds(t * BA, BA), :]
        cum = base_ref[...] + jnp.dot(tri_ref[...], oh2,
                                      preferred_element_type=jnp.float32)
        r1_ref[...] = jnp.sum(oh2 * cum, axis=-1, keepdims=True)
        base_ref[...] += jnp.sum(oh2, axis=0, keepdims=True)

    @pl.when(i == 2 * NTA - 1)
    def _finish():
        counts = base_ref[...]
        cnt_ref[...] = counts
        mean_prob = sump_ref[...] / N
        load_frac = counts / (N * K)
        loss_ref[...] = (BALANCE_WEIGHT * E
                         * jnp.sum(mean_prob * load_frac,
                                   axis=-1, keepdims=True))


def _run_router(h, csi, rwh, rwc, rb, tri):
    return pl.pallas_call(
        _router_kernel,
        grid=(2 * NTA,),
        in_specs=[
            pl.BlockSpec((BA, D), lambda i: (jnp.minimum(i, NTA - 1), 0)),
            pl.BlockSpec((BA, DCSI), lambda i: (jnp.minimum(i, NTA - 1), 0)),
            pl.BlockSpec((D, E), lambda i: (0, 0)),
            pl.BlockSpec((DCSI, E), lambda i: (0, 0)),
            pl.BlockSpec((1, E), lambda i: (0, 0)),
            pl.BlockSpec((BA, BA), lambda i: (0, 0)),
        ],
        out_specs=[
            pl.BlockSpec((BA, 1), lambda i: (jnp.minimum(i, NTA - 1), 0)),
            pl.BlockSpec((BA, 1), lambda i: (jnp.minimum(i, NTA - 1), 0)),
            pl.BlockSpec((BA, 1), lambda i: (jnp.minimum(i, NTA - 1), 0)),
            pl.BlockSpec((BA, 1), lambda i: (jnp.minimum(i, NTA - 1), 0)),
            pl.BlockSpec((BA, 1), lambda i: (jnp.minimum(i, NTA - 1), 0)),
            pl.BlockSpec((BA, 1), lambda i: (jnp.maximum(i - NTA, 0), 0)),
            pl.BlockSpec((1, E), lambda i: (0, 0)),
            pl.BlockSpec((1, 1), lambda i: (0, 0)),
            pl.BlockSpec((BA, D), lambda i: (jnp.minimum(i, NTA - 1), 0)),
        ],
        out_shape=[
            jax.ShapeDtypeStruct((N, 1), jnp.int32),
            jax.ShapeDtypeStruct((N, 1), jnp.int32),
            jax.ShapeDtypeStruct((N, 1), jnp.float32),
            jax.ShapeDtypeStruct((N, 1), jnp.float32),
            jax.ShapeDtypeStruct((N, 1), jnp.float32),
            jax.ShapeDtypeStruct((N, 1), jnp.float32),
            jax.ShapeDtypeStruct((1, E), jnp.float32),
            jax.ShapeDtypeStruct((1, 1), jnp.float32),
            jax.ShapeDtypeStruct((N, D), jnp.bfloat16),
        ],
        scratch_shapes=[
            pltpu.VMEM((N, E), jnp.float32),
            pltpu.VMEM((1, E), jnp.float32),
            pltpu.VMEM((1, E), jnp.float32),
        ],
    )(h, csi, rwh, rwc, rb, tri)


def _route_kernel(cnt_ref, e0_ref, e1_ref, r0_ref, r1_ref,
                  dst0_ref, dst1_ref, bexp_ref, offp_ref):
    i = pl.program_id(0)

    @pl.when(i == 0)
    def _offsets():
        counts = cnt_ref[...]
        nbpad = jnp.floor((counts + (BM - 1)) / BM) * BM
        ar = jax.lax.broadcasted_iota(jnp.int32, (E, E), 0)
        ac = jax.lax.broadcasted_iota(jnp.int32, (E, E), 1)
        tstrict = (ar < ac).astype(jnp.float32)
        offp = jnp.dot(nbpad, tstrict, preferred_element_type=jnp.float32)
        offp_ref[...] = offp
        cum_incl = offp + nbpad
        srows = (jax.lax.broadcasted_iota(jnp.int32, (NB, E), 0)
                 * BM).astype(jnp.float32)
        nfull = jnp.sum((cum_incl <= srows).astype(jnp.float32),
                        axis=-1, keepdims=True)
        bexp_ref[...] = jnp.minimum(nfull, E - 1).astype(jnp.int32)

    lane = jax.lax.broadcasted_iota(jnp.int32, (BTB, E), 1)
    offp = offp_ref[...]
    off0 = jnp.sum(jnp.where(lane == e0_ref[...], offp, 0.0),
                   axis=-1, keepdims=True)
    off1 = jnp.sum(jnp.where(lane == e1_ref[...], offp, 0.0),
                   axis=-1, keepdims=True)
    dst0_ref[...] = (off0 + r0_ref[...]).astype(jnp.int32)
    dst1_ref[...] = (off1 + r1_ref[...]).astype(jnp.int32)


def _run_route(cnt, e0, e1, r0, r1):
    return pl.pallas_call(
        _route_kernel,
        grid=(NTB,),
        in_specs=[
            pl.BlockSpec((1, E), lambda i: (0, 0)),
            pl.BlockSpec((BTB, 1), lambda i: (i, 0)),
            pl.BlockSpec((BTB, 1), lambda i: (i, 0)),
            pl.BlockSpec((BTB, 1), lambda i: (i, 0)),
            pl.BlockSpec((BTB, 1), lambda i: (i, 0)),
        ],
        out_specs=[
            pl.BlockSpec((BTB, 1), lambda i: (i, 0)),
            pl.BlockSpec((BTB, 1), lambda i: (i, 0)),
            pl.BlockSpec((NB, 1), lambda i: (0, 0)),
        ],
        out_shape=[
            jax.ShapeDtypeStruct((N, 1), jnp.int32),
            jax.ShapeDtypeStruct((N, 1), jnp.int32),
            jax.ShapeDtypeStruct((NB, 1), jnp.int32),
        ],
        scratch_shapes=[pltpu.VMEM((1, E), jnp.float32)],
    )(cnt, e0, e1, r0, r1)


def _run_scatter(dstp, gp):
    mesh = plsc.VectorSubcoreMesh(core_axis_name="c", subcore_axis_name="s")

    @functools.partial(
        pl.kernel, mesh=mesh,
        compiler_params=pltpu.CompilerParams(needs_layout_passes=False),
        out_type=[
            jax.ShapeDtypeStruct((M,), jnp.int32),
            jax.ShapeDtypeStruct((M,), jnp.float32),
        ],
        scratch_types=[
            pltpu.VMEM((P,), jnp.int32),
            pltpu.VMEM((P,), jnp.float32),
            pltpu.VMEM((M,), jnp.int32),
            pltpu.VMEM((M,), jnp.float32),
        ],
    )
    def kc(dstp_hbm, gp_hbm, src_hbm, gate_hbm, dst_v, g_v, src_v, gate_v):
        w = lax.axis_index("s") * 2 + lax.axis_index("c")

        @pl.when(w == 0)
        def _scatter_all():
            pltpu.sync_copy(dstp_hbm, dst_v)
            pltpu.sync_copy(gp_hbm, g_v)

            def body(j, _):
                idx16 = dst_v[pl.ds(j * L, L)]
                tok16 = (lax.iota(jnp.int32, L) + j * L) & (N - 1)
                plsc.store_scatter(src_v, [idx16], tok16)
                plsc.store_scatter(gate_v, [idx16], g_v[pl.ds(j * L, L)])
                return 0

            lax.fori_loop(0, P // L, body, 0)
            pltpu.sync_copy(src_v, src_hbm)
            pltpu.sync_copy(gate_v, gate_hbm)

    return kc(dstp, gp)


_RW = M // NWORK
_CH = 64
_NCH = _RW // _CH


def _run_gather_x(src, hbf_i):
    mesh = plsc.VectorSubcoreMesh(core_axis_name="c", subcore_axis_name="s")

    @functools.partial(
        pl.kernel, mesh=mesh,
        compiler_params=pltpu.CompilerParams(needs_layout_passes=False),
        out_type=jax.ShapeDtypeStruct((M, HD), jnp.int32),
        scratch_types=[
            pltpu.VMEM((_CH,), jnp.int32),
            pltpu.VMEM((_CH,), jnp.int32),
            pltpu.VMEM((_CH, HD), jnp.int32),
            pltpu.VMEM((_CH, HD), jnp.int32),
            pltpu.SemaphoreType.DMA,
            pltpu.SemaphoreType.DMA,
        ],
    )
    def kd(src_hbm, h_hbm, x_hbm, idx_a, idx_b, rows_a, rows_b, sem_a, sem_b):
        w = lax.axis_index("s") * 2 + lax.axis_index("c")
        base = w * _RW
        idxs = [idx_a, idx_b]
        rows = [rows_a, rows_b]
        sems = [sem_a, sem_b]

        def fetch(c, buf):
            idx_v, rows_v, sem = idxs[buf], rows[buf], sems[buf]
            pltpu.sync_copy(src_hbm.at[pl.ds(base + c * _CH, _CH)], idx_v)
            for s in range(_CH // L):
                v = idx_v[pl.ds(s * L, L)]
                idx_v[pl.ds(s * L, L)] = jnp.clip(v, 0, N - 1)
            return pltpu.async_copy(h_hbm.at[idx_v], rows_v, sem)

        cp = fetch(0, 0)
        for c in range(_NCH):
            nxt = None
            if c + 1 < _NCH:
                nxt = fetch(c + 1, (c + 1) % 2)
            cp.wait()
            pltpu.sync_copy(rows[c % 2], x_hbm.at[pl.ds(base + c * _CH, _CH)])
            cp = nxt

    return kd(src, hbf_i)


def _mlp_kernel(bexp_ref, x_ref, w1_ref, b1_ref, w2_ref, b2_ref, gate_ref,
                y_ref):
    x = x_ref[...].astype(jnp.float32)
    hid = (jnp.dot(x, w1_ref[0],
                   preferred_element_type=jnp.float32) + b1_ref[0])
    hid = jax.nn.gelu(hid)
    y = (jnp.dot(hid, w2_ref[0],
                 preferred_element_type=jnp.float32) + b2_ref[0])
    y_ref[...] = (y * gate_ref[...]).astype(jnp.bfloat16)


def _run_mlp(bexp, x_s, w1, b1, w2, b2, gate_s):
    grid_spec = pltpu.PrefetchScalarGridSpec(
        num_scalar_prefetch=1,
        grid=(NB,),
        in_specs=[
            pl.BlockSpec((BM, D), lambda i, be: (i, 0)),
            pl.BlockSpec((1, D, DFF), lambda i, be: (be[i], 0, 0)),
            pl.BlockSpec((1, 1, DFF), lambda i, be: (be[i], 0, 0)),
            pl.BlockSpec((1, DFF, D), lambda i, be: (be[i], 0, 0)),
            pl.BlockSpec((1, 1, D), lambda i, be: (be[i], 0, 0)),
            pl.BlockSpec((BM, 1), lambda i, be: (i, 0)),
        ],
        out_specs=pl.BlockSpec((BM, D), lambda i, be: (i, 0)),
    )
    return pl.pallas_call(
        _mlp_kernel,
        grid_spec=grid_spec,
        out_shape=jax.ShapeDtypeStruct((M, D), jnp.bfloat16),
    )(bexp, x_s, w1, b1[:, None, :], w2, b2[:, None, :], gate_s)


_TOKW = N // NWORK
_TCH = 64
_NTCH = _TOKW // _TCH


def _run_gather_y(dst0, dst1, y_i):
    mesh = plsc.VectorSubcoreMesh(core_axis_name="c", subcore_axis_name="s")

    @functools.partial(
        pl.kernel, mesh=mesh,
        compiler_params=pltpu.CompilerParams(needs_layout_passes=False),
        out_type=[
            jax.ShapeDtypeStruct((N, HD), jnp.int32),
            jax.ShapeDtypeStruct((N, HD), jnp.int32),
        ],
        scratch_types=[
            pltpu.VMEM((_TCH,), jnp.int32),
            pltpu.VMEM((_TCH,), jnp.int32),
            pltpu.VMEM((_TCH, HD), jnp.int32),
            pltpu.VMEM((_TCH, HD), jnp.int32),
            pltpu.VMEM((_TCH, HD), jnp.int32),
            pltpu.VMEM((_TCH, HD), jnp.int32),
            pltpu.SemaphoreType.DMA,
            pltpu.SemaphoreType.DMA,
        ],
    )
    def kf(d0_hbm, d1_hbm, y_hbm, y0_out, y1_out,
           i0_v, i1_v, a0_v, a1_v, b0_v, b1_v, sem0, sem1):
        w = lax.axis_index("s") * 2 + lax.axis_index("c")
        bufs = [(a0_v, a1_v), (b0_v, b1_v)]

        def fetch(c, buf):
            y0_v, y1_v = bufs[buf]
            tb = w * _TOKW + c * _TCH
            pltpu.sync_copy(d0_hbm.at[pl.ds(tb, _TCH)], i0_v)
            pltpu.sync_copy(d1_hbm.at[pl.ds(tb, _TCH)], i1_v)
            cp0 = pltpu.async_copy(y_hbm.at[i0_v], y0_v, sem0)
            cp1 = pltpu.async_copy(y_hbm.at[i1_v], y1_v, sem1)
            return cp0, cp1

        cps = fetch(0, 0)
        for c in range(_NTCH):
            nxt = None
            if c + 1 < _NTCH:
                nxt = fetch(c + 1, (c + 1) % 2)
            cps[0].wait()
            cps[1].wait()
            tb = w * _TOKW + c * _TCH
            y0_v, y1_v = bufs[c % 2]
            pltpu.sync_copy(y0_v, y0_out.at[pl.ds(tb, _TCH)])
            pltpu.sync_copy(y1_v, y1_out.at[pl.ds(tb, _TCH)])
            cps = nxt

    return kf(dst0, dst1, y_i)


_BTG = 2048


def _add_kernel(h_ref, y0_ref, y1_ref, out_ref):
    out_ref[...] = (h_ref[...]
                    + y0_ref[...].astype(jnp.float32)
                    + y1_ref[...].astype(jnp.float32))


def _run_add(h, y0, y1):
    return pl.pallas_call(
        _add_kernel,
        grid=(N // _BTG,),
        in_specs=[
            pl.BlockSpec((_BTG, D), lambda i: (i, 0)),
            pl.BlockSpec((_BTG, D), lambda i: (i, 0)),
            pl.BlockSpec((_BTG, D), lambda i: (i, 0)),
        ],
        out_specs=pl.BlockSpec((_BTG, D), lambda i: (i, 0)),
        out_shape=jax.ShapeDtypeStruct((N, D), jnp.float32),
    )(h, y0, y1)


_TRI = np.tril(np.ones((BA, BA), np.float32), -1)


@jax.jit
def kernel(h, router_in, router_w, router_b, w1, b1, w2, b2):
    csi = router_in[:, -DCSI:]
    rwh = router_w[:D, :]
    rwc = router_w[D:, :]
    rb = router_b[None, :]
    tri = jnp.asarray(_TRI)

    (e0, e1, g0, g1, r0, r1, cnt, loss, hbf) = _run_router(
        h, csi, rwh, rwc, rb, tri)
    dst0, dst1, bexp = _run_route(cnt, e0, e1, r0, r1)

    dst0_f = dst0.reshape(N)
    dst1_f = dst1.reshape(N)
    dstp = jnp.concatenate([dst0_f, dst1_f])
    gp = jnp.concatenate([g0.reshape(N), g1.reshape(N)])

    src, gate_s = _run_scatter(dstp, gp)
    x_i = _run_gather_x(src, _bf_to_i32(hbf))
    x_s = _i32_to_bf(x_i)
    y_s = _run_mlp(bexp.reshape(NB), x_s, w1, b1, w2, b2,
                   gate_s.reshape(M, 1))
    y0i, y1i = _run_gather_y(dst0_f, dst1_f, _bf_to_i32(y_s))
    out = _run_add(h, _i32_to_bf(y0i), _i32_to_bf(y1i))
    return out, loss[0, 0]


# speedup vs baseline: 3.7067x; 3.7067x over previous
import functools

import jax
import jax.numpy as jnp
import numpy as np
from jax import lax
from jax.experimental import pallas as pl
from jax.experimental.pallas import tpu as pltpu
from jax.experimental.pallas import tpu_sc as plsc

N = 8192
D = 768
E = 8
K = 2
DCSI = 10
DFF = 768
BALANCE_WEIGHT = 0.5

P = N * K
BA = 512
NTA = N // BA
BM = 256
M = P + E * BM
NB = M // BM
BTB = 2048
NTB = N // BTB

NWORK = 32
L = 16
HD = D // 2


_HIMASK = -65536


def _pack_bf16(lo_f32, hi_f32):
    lob = jax.lax.bitcast_convert_type(
        lo_f32.astype(jnp.bfloat16).astype(jnp.float32), jnp.int32)
    hib = jax.lax.bitcast_convert_type(
        hi_f32.astype(jnp.bfloat16).astype(jnp.float32), jnp.int32)
    return jax.lax.shift_right_logical(lob, 16) | (hib & _HIMASK)


def _unpack_lo(pk):
    return jax.lax.bitcast_convert_type(pk << 16, jnp.float32)


def _unpack_hi(pk):
    return jax.lax.bitcast_convert_type(pk & _HIMASK, jnp.float32)


def _router_kernel(h_ref, csi_ref, rwh_ref, rwc_ref, rb_ref, tri_ref,
                   e0_ref, e1_ref, g0_ref, g1_ref, r0_ref, r1_ref,
                   cnt_ref, loss_ref, hbf_ref,
                   oh2_ref, base_ref, sump_ref):
    i = pl.program_id(0)

    @pl.when(i == 0)
    def _init():
        base_ref[...] = jnp.zeros_like(base_ref)
        sump_ref[...] = jnp.zeros_like(sump_ref)

    @pl.when(i < NTA)
    def _phase1():
        t = i
        logits = (jnp.dot(h_ref[...], rwh_ref[...],
                          preferred_element_type=jnp.float32)
                  + jnp.dot(csi_ref[...], rwc_ref[...],
                            preferred_element_type=jnp.float32)
                  + rb_ref[...])
        mx = jnp.max(logits, axis=-1, keepdims=True)
        ex = jnp.exp(logits - mx)
        probs = ex / jnp.sum(ex, axis=-1, keepdims=True)
        lane = jax.lax.broadcasted_iota(jnp.int32, (BA, E), 1)
        v1 = jnp.max(probs, axis=-1, keepdims=True)
        e1 = jnp.min(jnp.where(probs == v1, lane, E), axis=-1, keepdims=True)
        masked = jnp.where(lane == e1, -jnp.inf, probs)
        v2 = jnp.max(masked, axis=-1, keepdims=True)
        e2 = jnp.min(jnp.where(masked == v2, lane, E), axis=-1, keepdims=True)
        denom = v1 + v2
        oh1 = (lane == e1).astype(jnp.float32)
        oh2 = (lane == e2).astype(jnp.float32)

        e0_ref[...] = e1
        e1_ref[...] = e2
        g0_ref[...] = v1 / denom
        g1_ref[...] = v2 / denom
        hb = h_ref[...]
        hbf_ref[...] = _pack_bf16(hb[:, :HD], hb[:, HD:])
        oh2_ref[pl.ds(t * BA, BA), :] = oh2

        cum = base_ref[...] + jnp.dot(tri_ref[...], oh1,
                                      preferred_element_type=jnp.float32)
        r0_ref[...] = jnp.sum(oh1 * cum, axis=-1, keepdims=True)
        base_ref[...] += jnp.sum(oh1, axis=0, keepdims=True)
        sump_ref[...] += jnp.sum(probs, axis=0, keepdims=True)

    @pl.when(i >= NTA)
    def _phase2():
        t = i - NTA
        oh2 = oh2_ref[pl.ds(t * BA, BA), :]
        cum = base_ref[...] + jnp.dot(tri_ref[...], oh2,
                                      preferred_element_type=jnp.float32)
        r1_ref[...] = jnp.sum(oh2 * cum, axis=-1, keepdims=True)
        base_ref[...] += jnp.sum(oh2, axis=0, keepdims=True)

    @pl.when(i == 2 * NTA - 1)
    def _finish():
        counts = base_ref[...]
        cnt_ref[...] = counts
        mean_prob = sump_ref[...] / N
        load_frac = counts / (N * K)
        loss_ref[...] = (BALANCE_WEIGHT * E
                         * jnp.sum(mean_prob * load_frac,
                                   axis=-1, keepdims=True))


def _run_router(h, csi, rwh, rwc, rb, tri):
    return pl.pallas_call(
        _router_kernel,
        grid=(2 * NTA,),
        in_specs=[
            pl.BlockSpec((BA, D), lambda i: (jnp.minimum(i, NTA - 1), 0)),
            pl.BlockSpec((BA, DCSI), lambda i: (jnp.minimum(i, NTA - 1), 0)),
            pl.BlockSpec((D, E), lambda i: (0, 0)),
            pl.BlockSpec((DCSI, E), lambda i: (0, 0)),
            pl.BlockSpec((1, E), lambda i: (0, 0)),
            pl.BlockSpec((BA, BA), lambda i: (0, 0)),
        ],
        out_specs=[
            pl.BlockSpec((BA, 1), lambda i: (jnp.minimum(i, NTA - 1), 0)),
            pl.BlockSpec((BA, 1), lambda i: (jnp.minimum(i, NTA - 1), 0)),
            pl.BlockSpec((BA, 1), lambda i: (jnp.minimum(i, NTA - 1), 0)),
            pl.BlockSpec((BA, 1), lambda i: (jnp.minimum(i, NTA - 1), 0)),
            pl.BlockSpec((BA, 1), lambda i: (jnp.minimum(i, NTA - 1), 0)),
            pl.BlockSpec((BA, 1), lambda i: (jnp.maximum(i - NTA, 0), 0)),
            pl.BlockSpec((1, E), lambda i: (0, 0)),
            pl.BlockSpec((1, 1), lambda i: (0, 0)),
            pl.BlockSpec((BA, HD), lambda i: (jnp.minimum(i, NTA - 1), 0)),
        ],
        out_shape=[
            jax.ShapeDtypeStruct((N, 1), jnp.int32),
            jax.ShapeDtypeStruct((N, 1), jnp.int32),
            jax.ShapeDtypeStruct((N, 1), jnp.float32),
            jax.ShapeDtypeStruct((N, 1), jnp.float32),
            jax.ShapeDtypeStruct((N, 1), jnp.float32),
            jax.ShapeDtypeStruct((N, 1), jnp.float32),
            jax.ShapeDtypeStruct((1, E), jnp.float32),
            jax.ShapeDtypeStruct((1, 1), jnp.float32),
            jax.ShapeDtypeStruct((N, HD), jnp.int32),
        ],
        scratch_shapes=[
            pltpu.VMEM((N, E), jnp.float32),
            pltpu.VMEM((1, E), jnp.float32),
            pltpu.VMEM((1, E), jnp.float32),
        ],
    )(h, csi, rwh, rwc, rb, tri)


def _route_kernel(cnt_ref, e0_ref, e1_ref, r0_ref, r1_ref,
                  dst0_ref, dst1_ref, bexp_ref, offp_ref):
    i = pl.program_id(0)

    @pl.when(i == 0)
    def _offsets():
        counts = cnt_ref[...]
        nbpad = jnp.floor((counts + (BM - 1)) / BM) * BM
        ar = jax.lax.broadcasted_iota(jnp.int32, (E, E), 0)
        ac = jax.lax.broadcasted_iota(jnp.int32, (E, E), 1)
        tstrict = (ar < ac).astype(jnp.float32)
        offp = jnp.dot(nbpad, tstrict, preferred_element_type=jnp.float32)
        offp_ref[...] = offp
        cum_incl = offp + nbpad
        srows = (jax.lax.broadcasted_iota(jnp.int32, (NB, E), 0)
                 * BM).astype(jnp.float32)
        nfull = jnp.sum((cum_incl <= srows).astype(jnp.float32),
                        axis=-1, keepdims=True)
        bexp_ref[...] = jnp.minimum(nfull, E - 1).astype(jnp.int32)

    lane = jax.lax.broadcasted_iota(jnp.int32, (BTB, E), 1)
    offp = offp_ref[...]
    off0 = jnp.sum(jnp.where(lane == e0_ref[...], offp, 0.0),
                   axis=-1, keepdims=True)
    off1 = jnp.sum(jnp.where(lane == e1_ref[...], offp, 0.0),
                   axis=-1, keepdims=True)
    dst0_ref[...] = (off0 + r0_ref[...]).astype(jnp.int32)
    dst1_ref[...] = (off1 + r1_ref[...]).astype(jnp.int32)


def _run_route(cnt, e0, e1, r0, r1):
    return pl.pallas_call(
        _route_kernel,
        grid=(NTB,),
        in_specs=[
            pl.BlockSpec((1, E), lambda i: (0, 0)),
            pl.BlockSpec((BTB, 1), lambda i: (i, 0)),
            pl.BlockSpec((BTB, 1), lambda i: (i, 0)),
            pl.BlockSpec((BTB, 1), lambda i: (i, 0)),
            pl.BlockSpec((BTB, 1), lambda i: (i, 0)),
        ],
        out_specs=[
            pl.BlockSpec((BTB, 1), lambda i: (i, 0)),
            pl.BlockSpec((BTB, 1), lambda i: (i, 0)),
            pl.BlockSpec((NB, 1), lambda i: (0, 0)),
        ],
        out_shape=[
            jax.ShapeDtypeStruct((N, 1), jnp.int32),
            jax.ShapeDtypeStruct((N, 1), jnp.int32),
            jax.ShapeDtypeStruct((NB, 1), jnp.int32),
        ],
        scratch_shapes=[pltpu.VMEM((1, E), jnp.float32)],
    )(cnt, e0, e1, r0, r1)


def _run_scatter(dstp, gp):
    mesh = plsc.VectorSubcoreMesh(core_axis_name="c", subcore_axis_name="s")

    @functools.partial(
        pl.kernel, mesh=mesh,
        compiler_params=pltpu.CompilerParams(needs_layout_passes=False),
        out_type=[
            jax.ShapeDtypeStruct((M,), jnp.int32),
            jax.ShapeDtypeStruct((M,), jnp.float32),
        ],
        scratch_types=[
            pltpu.VMEM((P,), jnp.int32),
            pltpu.VMEM((P,), jnp.float32),
            pltpu.VMEM((M,), jnp.int32),
            pltpu.VMEM((M,), jnp.float32),
        ],
    )
    def kc(dstp_hbm, gp_hbm, src_hbm, gate_hbm, dst_v, g_v, src_v, gate_v):
        w = lax.axis_index("s") * 2 + lax.axis_index("c")

        @pl.when(w == 0)
        def _scatter_all():
            pltpu.sync_copy(dstp_hbm, dst_v)
            pltpu.sync_copy(gp_hbm, g_v)

            def body(j, _):
                idx16 = dst_v[pl.ds(j * L, L)]
                tok16 = (lax.iota(jnp.int32, L) + j * L) & (N - 1)
                plsc.store_scatter(src_v, [idx16], tok16)
                plsc.store_scatter(gate_v, [idx16], g_v[pl.ds(j * L, L)])
                return 0

            lax.fori_loop(0, P // L, body, 0)
            pltpu.sync_copy(src_v, src_hbm)
            pltpu.sync_copy(gate_v, gate_hbm)

    return kc(dstp, gp)


_RW = M // NWORK
_CH = 64
_NCH = _RW // _CH


def _run_gather_x(src, hbf_i):
    mesh = plsc.VectorSubcoreMesh(core_axis_name="c", subcore_axis_name="s")

    @functools.partial(
        pl.kernel, mesh=mesh,
        compiler_params=pltpu.CompilerParams(needs_layout_passes=False),
        out_type=jax.ShapeDtypeStruct((M, HD), jnp.int32),
        scratch_types=[
            pltpu.VMEM((_CH,), jnp.int32),
            pltpu.VMEM((_CH,), jnp.int32),
            pltpu.VMEM((_CH, HD), jnp.int32),
            pltpu.VMEM((_CH, HD), jnp.int32),
            pltpu.SemaphoreType.DMA,
            pltpu.SemaphoreType.DMA,
        ],
    )
    def kd(src_hbm, h_hbm, x_hbm, idx_a, idx_b, rows_a, rows_b, sem_a, sem_b):
        w = lax.axis_index("s") * 2 + lax.axis_index("c")
        base = w * _RW
        idxs = [idx_a, idx_b]
        rows = [rows_a, rows_b]
        sems = [sem_a, sem_b]

        def fetch(c, buf):
            idx_v, rows_v, sem = idxs[buf], rows[buf], sems[buf]
            pltpu.sync_copy(src_hbm.at[pl.ds(base + c * _CH, _CH)], idx_v)
            for s in range(_CH // L):
                v = idx_v[pl.ds(s * L, L)]
                idx_v[pl.ds(s * L, L)] = jnp.clip(v, 0, N - 1)
            return pltpu.async_copy(h_hbm.at[idx_v], rows_v, sem)

        cp = fetch(0, 0)
        for c in range(_NCH):
            nxt = None
            if c + 1 < _NCH:
                nxt = fetch(c + 1, (c + 1) % 2)
            cp.wait()
            pltpu.sync_copy(rows[c % 2], x_hbm.at[pl.ds(base + c * _CH, _CH)])
            cp = nxt

    return kd(src, hbf_i)


def _mlp_kernel(bexp_ref, x_ref, w1a_ref, w1b_ref, b1_ref, w2_ref, b2_ref,
                gate_ref, y_ref):
    pk = x_ref[...]
    xlo = _unpack_lo(pk)
    xhi = _unpack_hi(pk)
    hid = (jnp.dot(xlo, w1a_ref[0], preferred_element_type=jnp.float32)
           + jnp.dot(xhi, w1b_ref[0], preferred_element_type=jnp.float32)
           + b1_ref[0])
    hid = jax.nn.gelu(hid)
    y = (jnp.dot(hid, w2_ref[0],
                 preferred_element_type=jnp.float32) + b2_ref[0])
    y = y * gate_ref[...]
    y_ref[...] = _pack_bf16(y[:, :HD], y[:, HD:])


def _run_mlp(bexp, x_s, w1, b1, w2, b2, gate_s):
    grid_spec = pltpu.PrefetchScalarGridSpec(
        num_scalar_prefetch=1,
        grid=(NB,),
        in_specs=[
            pl.BlockSpec((BM, HD), lambda i, be: (i, 0)),
            pl.BlockSpec((1, HD, DFF), lambda i, be: (be[i], 0, 0)),
            pl.BlockSpec((1, HD, DFF), lambda i, be: (be[i], 1, 0)),
            pl.BlockSpec((1, 1, DFF), lambda i, be: (be[i], 0, 0)),
            pl.BlockSpec((1, DFF, D), lambda i, be: (be[i], 0, 0)),
            pl.BlockSpec((1, 1, D), lambda i, be: (be[i], 0, 0)),
            pl.BlockSpec((BM, 1), lambda i, be: (i, 0)),
        ],
        out_specs=pl.BlockSpec((BM, HD), lambda i, be: (i, 0)),
    )
    return pl.pallas_call(
        _mlp_kernel,
        grid_spec=grid_spec,
        out_shape=jax.ShapeDtypeStruct((M, HD), jnp.int32),
    )(bexp, x_s, w1, w1, b1[:, None, :], w2, b2[:, None, :], gate_s)


_TOKW = N // NWORK
_TCH = 64
_NTCH = _TOKW // _TCH


def _run_gather_y(dst0, dst1, y_i):
    mesh = plsc.VectorSubcoreMesh(core_axis_name="c", subcore_axis_name="s")

    @functools.partial(
        pl.kernel, mesh=mesh,
        compiler_params=pltpu.CompilerParams(needs_layout_passes=False),
        out_type=[
            jax.ShapeDtypeStruct((N, HD), jnp.int32),
            jax.ShapeDtypeStruct((N, HD), jnp.int32),
        ],
        scratch_types=[
            pltpu.VMEM((_TCH,), jnp.int32),
            pltpu.VMEM((_TCH,), jnp.int32),
            pltpu.VMEM((_TCH, HD), jnp.int32),
            pltpu.VMEM((_TCH, HD), jnp.int32),
            pltpu.VMEM((_TCH, HD), jnp.int32),
            pltpu.VMEM((_TCH, HD), jnp.int32),
            pltpu.SemaphoreType.DMA,
            pltpu.SemaphoreType.DMA,
        ],
    )
    def kf(d0_hbm, d1_hbm, y_hbm, y0_out, y1_out,
           i0_v, i1_v, a0_v, a1_v, b0_v, b1_v, sem0, sem1):
        w = lax.axis_index("s") * 2 + lax.axis_index("c")
        bufs = [(a0_v, a1_v), (b0_v, b1_v)]

        def fetch(c, buf):
            y0_v, y1_v = bufs[buf]
            tb = w * _TOKW + c * _TCH
            pltpu.sync_copy(d0_hbm.at[pl.ds(tb, _TCH)], i0_v)
            pltpu.sync_copy(d1_hbm.at[pl.ds(tb, _TCH)], i1_v)
            cp0 = pltpu.async_copy(y_hbm.at[i0_v], y0_v, sem0)
            cp1 = pltpu.async_copy(y_hbm.at[i1_v], y1_v, sem1)
            return cp0, cp1

        cps = fetch(0, 0)
        for c in range(_NTCH):
            nxt = None
            if c + 1 < _NTCH:
                nxt = fetch(c + 1, (c + 1) % 2)
            cps[0].wait()
            cps[1].wait()
            tb = w * _TOKW + c * _TCH
            y0_v, y1_v = bufs[c % 2]
            pltpu.sync_copy(y0_v, y0_out.at[pl.ds(tb, _TCH)])
            pltpu.sync_copy(y1_v, y1_out.at[pl.ds(tb, _TCH)])
            cps = nxt

    return kf(dst0, dst1, y_i)


_BTG = 2048


def _add_kernel(h_ref, y0_ref, y1_ref, out_ref):
    h = h_ref[...]
    p0 = y0_ref[...]
    p1 = y1_ref[...]
    out_ref[:, :HD] = h[:, :HD] + _unpack_lo(p0) + _unpack_lo(p1)
    out_ref[:, HD:] = h[:, HD:] + _unpack_hi(p0) + _unpack_hi(p1)


def _run_add(h, y0, y1):
    return pl.pallas_call(
        _add_kernel,
        grid=(N // _BTG,),
        in_specs=[
            pl.BlockSpec((_BTG, D), lambda i: (i, 0)),
            pl.BlockSpec((_BTG, HD), lambda i: (i, 0)),
            pl.BlockSpec((_BTG, HD), lambda i: (i, 0)),
        ],
        out_specs=pl.BlockSpec((_BTG, D), lambda i: (i, 0)),
        out_shape=jax.ShapeDtypeStruct((N, D), jnp.float32),
    )(h, y0, y1)


_TRI = np.tril(np.ones((BA, BA), np.float32), -1)


@jax.jit
def kernel(h, router_in, router_w, router_b, w1, b1, w2, b2):
    csi = router_in[:, -DCSI:]
    rwh = router_w[:D, :]
    rwc = router_w[D:, :]
    rb = router_b[None, :]
    tri = jnp.asarray(_TRI)

    (e0, e1, g0, g1, r0, r1, cnt, loss, hbf) = _run_router(
        h, csi, rwh, rwc, rb, tri)
    dst0, dst1, bexp = _run_route(cnt, e0, e1, r0, r1)

    dst0_f = dst0.reshape(N)
    dst1_f = dst1.reshape(N)
    dstp = jnp.concatenate([dst0_f, dst1_f])
    gp = jnp.concatenate([g0.reshape(N), g1.reshape(N)])

    src, gate_s = _run_scatter(dstp, gp)
    x_pk = _run_gather_x(src, hbf)
    y_pk = _run_mlp(bexp.reshape(NB), x_pk, w1, b1, w2, b2,
                    gate_s.reshape(M, 1))
    y0pk, y1pk = _run_gather_y(dst0_f, dst1_f, y_pk)
    out = _run_add(h, y0pk, y1pk)
    return out, loss[0, 0]
